# Initial kernel scaffold; baseline (speedup 1.0000x reference)
import jax
import jax.numpy as jnp
from jax.experimental import pallas as pl


def kernel(x, adj_mat, W1, a1_l, a1_r, W2, a2_l, a2_r):
    raise NotImplementedError("write your pallas kernel here")


# speedup vs baseline: 1.6241x; 1.6241x over previous
import jax
import jax.numpy as jnp
from jax.experimental import pallas as pl

_N = 2048
_F = 256
_NH = 8
_HD = 32
_C = 32
_RA = 512
_RB = 256
_RC = 512
_NEG = -1e9


def _proj_kernel(x_ref, w1_ref, w1h_ref, al_ref, ar_ref, gh_ref, el_ref, er_ref):
    x = x_ref[...]
    g = jnp.dot(x, w1_ref[...], preferred_element_type=jnp.float32)
    el_ref[...] = jnp.dot(g, al_ref[...], preferred_element_type=jnp.float32)
    er_ref[...] = jnp.dot(g, ar_ref[...], preferred_element_type=jnp.float32)
    for h in range(_NH):
        gh_ref[h] = jnp.dot(x, w1h_ref[h], preferred_element_type=jnp.float32)


def _attn1_kernel(el_ref, ert_ref, gh_ref, adj_ref, w2h_ref, a2_ref,
                  g2_ref, aux_ref):
    mask = adj_ref[...] != 0
    el = el_ref[...]
    ert = ert_ref[...]
    acc = jnp.zeros((el.shape[0], _C), jnp.float32)
    for h in range(_NH):
        s = el[:, h:h + 1] + ert[h:h + 1, :]
        s = jnp.where(s >= 0, s, 0.2 * s)
        s = jnp.where(mask, s, _NEG)
        m = jnp.max(s, axis=1, keepdims=True)
        p = jnp.exp(s - m)
        denom = jnp.sum(p, axis=1, keepdims=True)
        o = jnp.dot(p, gh_ref[h], preferred_element_type=jnp.float32) / denom
        o = jnp.where(o > 0, o, jnp.exp(o) - 1.0)
        acc = acc + jnp.dot(o, w2h_ref[h], preferred_element_type=jnp.float32)
    g2_ref[...] = acc
    aux_ref[...] = jnp.dot(acc, a2_ref[...], preferred_element_type=jnp.float32)


def _attn2_kernel(el2_ref, er2t_ref, g2_ref, adj_ref, out_ref):
    mask = adj_ref[...] != 0
    s = el2_ref[...] + er2t_ref[...]
    s = jnp.where(s >= 0, s, 0.2 * s)
    s = jnp.where(mask, s, _NEG)
    m = jnp.max(s, axis=1, keepdims=True)
    p = jnp.exp(s - m)
    denom = jnp.sum(p, axis=1, keepdims=True)
    out_ref[...] = jnp.dot(p, g2_ref[...],
                           preferred_element_type=jnp.float32) / denom


def kernel(x, adj_mat, W1, a1_l, a1_r, W2, a2_l, a2_r):
    f32 = jnp.float32
    adj = adj_mat.reshape(_N, _N).astype(jnp.int8)
    W1h = W1.reshape(_F, _NH, _HD).transpose(1, 0, 2)
    AL = jnp.kron(jnp.eye(_NH, dtype=f32), a1_l[:, None])
    AR = jnp.kron(jnp.eye(_NH, dtype=f32), a1_r[:, None])
    W2h = W2.reshape(_NH, _HD, _C)
    A2 = jnp.stack([a2_l, a2_r], axis=1)

    gh, el, er = pl.pallas_call(
        _proj_kernel,
        grid=(_N // _RA,),
        in_specs=[
            pl.BlockSpec((_RA, _F), lambda i: (i, 0)),
            pl.BlockSpec((_F, _F), lambda i: (0, 0)),
            pl.BlockSpec((_NH, _F, _HD), lambda i: (0, 0, 0)),
            pl.BlockSpec((_F, _NH), lambda i: (0, 0)),
            pl.BlockSpec((_F, _NH), lambda i: (0, 0)),
        ],
        out_specs=[
            pl.BlockSpec((_NH, _RA, _HD), lambda i: (0, i, 0)),
            pl.BlockSpec((_RA, _NH), lambda i: (i, 0)),
            pl.BlockSpec((_RA, _NH), lambda i: (i, 0)),
        ],
        out_shape=[
            jax.ShapeDtypeStruct((_NH, _N, _HD), f32),
            jax.ShapeDtypeStruct((_N, _NH), f32),
            jax.ShapeDtypeStruct((_N, _NH), f32),
        ],
    )(x, W1, W1h, AL, AR)

    g2, aux = pl.pallas_call(
        _attn1_kernel,
        grid=(_N // _RB,),
        in_specs=[
            pl.BlockSpec((_RB, _NH), lambda i: (i, 0)),
            pl.BlockSpec((_NH, _N), lambda i: (0, 0)),
            pl.BlockSpec((_NH, _N, _HD), lambda i: (0, 0, 0)),
            pl.BlockSpec((_RB, _N), lambda i: (i, 0)),
            pl.BlockSpec((_NH, _HD, _C), lambda i: (0, 0, 0)),
            pl.BlockSpec((_C, 2), lambda i: (0, 0)),
        ],
        out_specs=[
            pl.BlockSpec((_RB, _C), lambda i: (i, 0)),
            pl.BlockSpec((_RB, 2), lambda i: (i, 0)),
        ],
        out_shape=[
            jax.ShapeDtypeStruct((_N, _C), f32),
            jax.ShapeDtypeStruct((_N, 2), f32),
        ],
    )(el, er.T, gh, adj, W2h, A2)

    out = pl.pallas_call(
        _attn2_kernel,
        grid=(_N // _RC,),
        in_specs=[
            pl.BlockSpec((_RC, 1), lambda i: (i, 0)),
            pl.BlockSpec((1, _N), lambda i: (0, 0)),
            pl.BlockSpec((_N, _C), lambda i: (0, 0)),
            pl.BlockSpec((_RC, _N), lambda i: (i, 0)),
        ],
        out_specs=pl.BlockSpec((_RC, _C), lambda i: (i, 0)),
        out_shape=jax.ShapeDtypeStruct((_N, _C), f32),
    )(aux[:, 0:1], aux[:, 1:2].T, g2, adj)

    return out


# speedup vs baseline: 1.6880x; 1.0394x over previous
import jax
import jax.numpy as jnp
from jax.experimental import pallas as pl

_N = 2048
_F = 256
_NH = 8
_HD = 32
_C = 32
_RA = 512
_RB = 256
_RC = 512
_NEG = -1e9
_LOG2E = 1.4426950408889634


def _proj_kernel(x_ref, w1_ref, w1h_ref, al_ref, ar_ref, gh_ref, el_ref, er_ref):
    x = x_ref[...]
    g = jnp.dot(x, w1_ref[...], preferred_element_type=jnp.float32)
    el_ref[...] = jnp.dot(g, al_ref[...], preferred_element_type=jnp.float32)
    er_ref[...] = jnp.dot(g, ar_ref[...], preferred_element_type=jnp.float32)
    for h in range(_NH):
        gh_ref[h] = jnp.dot(x, w1h_ref[h], preferred_element_type=jnp.float32)


def _attn1_kernel(el_ref, ert_ref, gh_ref, adj_ref, w2h_ref, a2_ref,
                  g2_ref, aux_ref):
    mask = adj_ref[...] != 0
    el = el_ref[...]
    ert = ert_ref[...]
    acc = jnp.zeros((el.shape[0], _C), jnp.float32)
    for h in range(_NH):
        s = el[:, h:h + 1] + ert[h:h + 1, :]
        s = jnp.maximum(s, 0.2 * s)
        s = jnp.where(mask, s, _NEG)
        m = jnp.max(s, axis=1, keepdims=True)
        p = jnp.exp2(s - m)
        og = jnp.dot(p, gh_ref[h], preferred_element_type=jnp.float32)
        o = og[:, :_HD] / og[:, _HD:_HD + 1]
        o = jnp.where(o > 0, o, jnp.exp(o) - 1.0)
        acc = acc + jnp.dot(o, w2h_ref[h], preferred_element_type=jnp.float32)
    g2_ref[...] = acc
    aux_ref[...] = jnp.dot(acc, a2_ref[...], preferred_element_type=jnp.float32)


def _attn2_kernel(el2_ref, er2t_ref, g2a_ref, adj_ref, out_ref):
    mask = adj_ref[...] != 0
    s = el2_ref[...] + er2t_ref[...]
    s = jnp.maximum(s, 0.2 * s)
    s = jnp.where(mask, s, _NEG)
    m = jnp.max(s, axis=1, keepdims=True)
    p = jnp.exp2(s - m)
    og = jnp.dot(p, g2a_ref[...], preferred_element_type=jnp.float32)
    out_ref[...] = og[:, :_C] / og[:, _C:_C + 1]


def kernel(x, adj_mat, W1, a1_l, a1_r, W2, a2_l, a2_r):
    f32 = jnp.float32
    adj = adj_mat.reshape(_N, _N).astype(jnp.int8)
    x_aug = jnp.concatenate([x, jnp.ones((_N, 1), f32)], axis=1)
    W1p = jnp.concatenate([W1, jnp.zeros((1, _F), f32)], axis=0)
    W1h = W1.reshape(_F, _NH, _HD).transpose(1, 0, 2)
    W1ha = jnp.zeros((_NH, _F + 1, _HD + 1), f32)
    W1ha = W1ha.at[:, :_F, :_HD].set(W1h).at[:, _F, _HD].set(1.0)
    AL = jnp.kron(jnp.eye(_NH, dtype=f32), a1_l[:, None]) * _LOG2E
    AR = jnp.kron(jnp.eye(_NH, dtype=f32), a1_r[:, None]) * _LOG2E
    W2h = W2.reshape(_NH, _HD, _C)
    A2 = jnp.stack([a2_l, a2_r], axis=1) * _LOG2E

    gh, el, er = pl.pallas_call(
        _proj_kernel,
        grid=(_N // _RA,),
        in_specs=[
            pl.BlockSpec((_RA, _F + 1), lambda i: (i, 0)),
            pl.BlockSpec((_F + 1, _F), lambda i: (0, 0)),
            pl.BlockSpec((_NH, _F + 1, _HD + 1), lambda i: (0, 0, 0)),
            pl.BlockSpec((_F, _NH), lambda i: (0, 0)),
            pl.BlockSpec((_F, _NH), lambda i: (0, 0)),
        ],
        out_specs=[
            pl.BlockSpec((_NH, _RA, _HD + 1), lambda i: (0, i, 0)),
            pl.BlockSpec((_RA, _NH), lambda i: (i, 0)),
            pl.BlockSpec((_RA, _NH), lambda i: (i, 0)),
        ],
        out_shape=[
            jax.ShapeDtypeStruct((_NH, _N, _HD + 1), f32),
            jax.ShapeDtypeStruct((_N, _NH), f32),
            jax.ShapeDtypeStruct((_N, _NH), f32),
        ],
    )(x_aug, W1p, W1ha, AL, AR)

    g2, aux = pl.pallas_call(
        _attn1_kernel,
        grid=(_N // _RB,),
        in_specs=[
            pl.BlockSpec((_RB, _NH), lambda i: (i, 0)),
            pl.BlockSpec((_NH, _N), lambda i: (0, 0)),
            pl.BlockSpec((_NH, _N, _HD + 1), lambda i: (0, 0, 0)),
            pl.BlockSpec((_RB, _N), lambda i: (i, 0)),
            pl.BlockSpec((_NH, _HD, _C), lambda i: (0, 0, 0)),
            pl.BlockSpec((_C, 2), lambda i: (0, 0)),
        ],
        out_specs=[
            pl.BlockSpec((_RB, _C), lambda i: (i, 0)),
            pl.BlockSpec((_RB, 2), lambda i: (i, 0)),
        ],
        out_shape=[
            jax.ShapeDtypeStruct((_N, _C), f32),
            jax.ShapeDtypeStruct((_N, 2), f32),
        ],
    )(el, er.T, gh, adj, W2h, A2)

    g2a = jnp.concatenate([g2, jnp.ones((_N, 1), f32)], axis=1)

    out = pl.pallas_call(
        _attn2_kernel,
        grid=(_N // _RC,),
        in_specs=[
            pl.BlockSpec((_RC, 1), lambda i: (i, 0)),
            pl.BlockSpec((1, _N), lambda i: (0, 0)),
            pl.BlockSpec((_N, _C + 1), lambda i: (0, 0)),
            pl.BlockSpec((_RC, _N), lambda i: (i, 0)),
        ],
        out_specs=pl.BlockSpec((_RC, _C), lambda i: (i, 0)),
        out_shape=jax.ShapeDtypeStruct((_N, _C), f32),
    )(aux[:, 0:1], aux[:, 1:2].T, g2a, adj)

    return out


# speedup vs baseline: 1.8151x; 1.0753x over previous
import jax
import jax.numpy as jnp
from jax.experimental import pallas as pl
from jax.experimental.pallas import tpu as pltpu

_N = 2048
_F = 256
_NH = 8
_HD = 32
_C = 32
_RB = 256
_RC = 512
_NEG = -1e9
_LOG2E = 1.4426950408889634


def _gat1_kernel(x_ref, xt_ref, w1h_ref, elm_ref, erm_ref, adj_ref, w2h_ref,
                 a2_ref, g2a_ref, aux_ref, gh_scr, el_scr, ert_scr):
    f32 = jnp.float32
    k = pl.program_id(0)

    @pl.when(k == 0)
    def _prologue():
        x = x_ref[...]
        el_scr[...] = jnp.dot(x, elm_ref[...], preferred_element_type=f32)
        ert_scr[...] = jnp.dot(erm_ref[...], xt_ref[...],
                               preferred_element_type=f32)
        ones = jnp.ones((_N, 1), f32)
        for h in range(_NH):
            gh = jnp.dot(x, w1h_ref[h], preferred_element_type=f32)
            gh_scr[h] = jnp.concatenate([gh, ones], axis=1)

    mask = adj_ref[...] != 0
    el = el_scr[pl.ds(k * _RB, _RB), :]
    ert = ert_scr[...]
    acc = jnp.zeros((_RB, _C), f32)
    for h in range(_NH):
        s = el[:, h:h + 1] + ert[h:h + 1, :]
        s = jnp.maximum(s, 0.2 * s)
        s = jnp.where(mask, s, _NEG)
        m = jnp.max(s, axis=1, keepdims=True)
        p = jnp.exp2(s - m)
        og = jnp.dot(p, gh_scr[h], preferred_element_type=f32)
        o = og[:, :_HD] / og[:, _HD:_HD + 1]
        o = jnp.where(o > 0, o, jnp.exp(o) - 1.0)
        acc = acc + jnp.dot(o, w2h_ref[h], preferred_element_type=f32)
    g2a_ref[...] = jnp.concatenate([acc, jnp.ones((_RB, 1), f32)], axis=1)
    aux_ref[...] = jnp.dot(acc, a2_ref[...], preferred_element_type=f32)


def _attn2_kernel(el2_ref, er2t_ref, g2a_ref, adj_ref, out_ref):
    mask = adj_ref[...] != 0
    s = el2_ref[...] + er2t_ref[...]
    s = jnp.maximum(s, 0.2 * s)
    s = jnp.where(mask, s, _NEG)
    m = jnp.max(s, axis=1, keepdims=True)
    p = jnp.exp2(s - m)
    og = jnp.dot(p, g2a_ref[...], preferred_element_type=jnp.float32)
    out_ref[...] = og[:, :_C] / og[:, _C:_C + 1]


def kernel(x, adj_mat, W1, a1_l, a1_r, W2, a2_l, a2_r):
    f32 = jnp.float32
    adj = adj_mat.reshape(_N, _N).astype(jnp.int8)
    xt = x.T
    W1h = W1.reshape(_F, _NH, _HD).transpose(1, 0, 2)
    AL = jnp.kron(jnp.eye(_NH, dtype=f32), a1_l[:, None]) * _LOG2E
    AR = jnp.kron(jnp.eye(_NH, dtype=f32), a1_r[:, None]) * _LOG2E
    ELM = W1 @ AL
    ERM = (W1 @ AR).T
    W2h = W2.reshape(_NH, _HD, _C)
    A2 = jnp.stack([a2_l, a2_r], axis=1) * _LOG2E

    g2a, aux = pl.pallas_call(
        _gat1_kernel,
        grid=(_N // _RB,),
        in_specs=[
            pl.BlockSpec((_N, _F), lambda i: (0, 0)),
            pl.BlockSpec((_F, _N), lambda i: (0, 0)),
            pl.BlockSpec((_NH, _F, _HD), lambda i: (0, 0, 0)),
            pl.BlockSpec((_F, _NH), lambda i: (0, 0)),
            pl.BlockSpec((_NH, _F), lambda i: (0, 0)),
            pl.BlockSpec((_RB, _N), lambda i: (i, 0)),
            pl.BlockSpec((_NH, _HD, _C), lambda i: (0, 0, 0)),
            pl.BlockSpec((_C, 2), lambda i: (0, 0)),
        ],
        out_specs=[
            pl.BlockSpec((_RB, _C + 1), lambda i: (i, 0)),
            pl.BlockSpec((_RB, 2), lambda i: (i, 0)),
        ],
        out_shape=[
            jax.ShapeDtypeStruct((_N, _C + 1), f32),
            jax.ShapeDtypeStruct((_N, 2), f32),
        ],
        scratch_shapes=[
            pltpu.VMEM((_NH, _N, _HD + 1), f32),
            pltpu.VMEM((_N, _NH), f32),
            pltpu.VMEM((_NH, _N), f32),
        ],
    )(x, xt, W1h, ELM, ERM, adj, W2h, A2)

    out = pl.pallas_call(
        _attn2_kernel,
        grid=(_N // _RC,),
        in_specs=[
            pl.BlockSpec((_RC, 1), lambda i: (i, 0)),
            pl.BlockSpec((1, _N), lambda i: (0, 0)),
            pl.BlockSpec((_N, _C + 1), lambda i: (0, 0)),
            pl.BlockSpec((_RC, _N), lambda i: (i, 0)),
        ],
        out_specs=pl.BlockSpec((_RC, _C), lambda i: (i, 0)),
        out_shape=jax.ShapeDtypeStruct((_N, _C), f32),
    )(aux[:, 0:1], aux[:, 1:2].T, g2a, adj)

    return out


# speedup vs baseline: 1.8310x; 1.0087x over previous
import jax
import jax.numpy as jnp
from jax.experimental import pallas as pl
from jax.experimental.pallas import tpu as pltpu

_N = 2048
_F = 256
_NH = 8
_HD = 32
_C = 32
_R = 256
_NB = _N // _R
_NEG = -1e9
_LOG2E = 1.4426950408889634


def _gat_kernel(x_ref, w1h_ref, elm_ref, erm_ref, adj_ref, w2h_ref, a2_ref,
                out_ref, gh_scr, el_scr, ert_scr, g2a_scr, aux_scr, auxt_scr):
    f32 = jnp.float32
    k = pl.program_id(0)

    @pl.when(k == 0)
    def _prologue():
        x = x_ref[...]
        el_scr[...] = jnp.dot(x, elm_ref[...], preferred_element_type=f32)
        er = jnp.dot(x, erm_ref[...], preferred_element_type=f32)
        ert_scr[...] = jnp.transpose(er)
        ones = jnp.ones((_N, 1), f32)
        for h in range(_NH):
            gh = jnp.dot(x, w1h_ref[h], preferred_element_type=f32)
            gh_scr[h] = jnp.concatenate([gh, ones], axis=1)

    mask = adj_ref[...]

    @pl.when(k < _NB)
    def _layer1():
        el = el_scr[pl.ds(k * _R, _R), :]
        ert = ert_scr[...]
        acc = jnp.zeros((_R, _C), f32)
        for h in range(_NH):
            s = el[:, h:h + 1] + ert[h:h + 1, :]
            s = jnp.maximum(s, 0.2 * s)
            s = jnp.where(mask, s, _NEG)
            m = jnp.max(s, axis=1, keepdims=True)
            p = jnp.exp2(s - m)
            og = jnp.dot(p, gh_scr[h], preferred_element_type=f32)
            o = og[:, :_HD] / og[:, _HD:_HD + 1]
            o = jnp.where(o > 0, o, jnp.exp(o) - 1.0)
            acc = acc + jnp.dot(o, w2h_ref[h], preferred_element_type=f32)
        g2a_scr[pl.ds(k * _R, _R), :] = jnp.concatenate(
            [acc, jnp.ones((_R, 1), f32)], axis=1)
        aux_scr[pl.ds(k * _R, _R), :] = jnp.dot(
            acc, a2_ref[...], preferred_element_type=f32)

    @pl.when(k == _NB)
    def _transpose_aux():
        auxt_scr[...] = jnp.transpose(aux_scr[...])

    @pl.when(k >= _NB)
    def _layer2():
        el2 = aux_scr[pl.ds((k - _NB) * _R, _R), 0:1]
        s = el2 + auxt_scr[1:2, :]
        s = jnp.maximum(s, 0.2 * s)
        s = jnp.where(mask, s, _NEG)
        m = jnp.max(s, axis=1, keepdims=True)
        p = jnp.exp2(s - m)
        og = jnp.dot(p, g2a_scr[...], preferred_element_type=f32)
        out_ref[...] = og[:, :_C] / og[:, _C:_C + 1]


def kernel(x, adj_mat, W1, a1_l, a1_r, W2, a2_l, a2_r):
    f32 = jnp.float32
    adj = adj_mat.reshape(_N, _N)
    W1h = W1.reshape(_F, _NH, _HD).transpose(1, 0, 2)
    AL = jnp.kron(jnp.eye(_NH, dtype=f32), a1_l[:, None]) * _LOG2E
    AR = jnp.kron(jnp.eye(_NH, dtype=f32), a1_r[:, None]) * _LOG2E
    ELM = W1 @ AL
    ERM = W1 @ AR
    W2h = W2.reshape(_NH, _HD, _C)
    A2 = jnp.stack([a2_l, a2_r], axis=1) * _LOG2E

    out = pl.pallas_call(
        _gat_kernel,
        grid=(2 * _NB,),
        in_specs=[
            pl.BlockSpec((_N, _F), lambda k: (0, 0)),
            pl.BlockSpec((_NH, _F, _HD), lambda k: (0, 0, 0)),
            pl.BlockSpec((_F, _NH), lambda k: (0, 0)),
            pl.BlockSpec((_F, _NH), lambda k: (0, 0)),
            pl.BlockSpec((_R, _N), lambda k: (jax.lax.rem(k, _NB), 0)),
            pl.BlockSpec((_NH, _HD, _C), lambda k: (0, 0, 0)),
            pl.BlockSpec((_C, 2), lambda k: (0, 0)),
        ],
        out_specs=pl.BlockSpec(
            (_R, _C), lambda k: (jnp.maximum(k - _NB, 0), 0)),
        out_shape=jax.ShapeDtypeStruct((_N, _C), f32),
        scratch_shapes=[
            pltpu.VMEM((_NH, _N, _HD + 1), f32),
            pltpu.VMEM((_N, _NH), f32),
            pltpu.VMEM((_NH, _N), f32),
            pltpu.VMEM((_N, _C + 1), f32),
            pltpu.VMEM((_N, 2), f32),
            pltpu.VMEM((2, _N), f32),
        ],
    )(x, W1h, ELM, ERM, adj, W2h, A2)

    return out
